# Initial kernel scaffold; baseline (speedup 1.0000x reference)
import jax
import jax.numpy as jnp
from jax.experimental import pallas as pl


def kernel(input, W):
    raise NotImplementedError("write your pallas kernel here")


# speedup vs baseline: 2.0432x; 2.0432x over previous
import functools

import jax
import jax.numpy as jnp
from jax.experimental import pallas as pl

_NUM_EMB = 1024
_EMB_DIM = 64
_BLOCK = 2048


def _vq_block_kernel(x_ref, w_ref, out_ref, loss_ref):
    x = x_ref[...]
    w = w_ref[...]
    b = x.shape[0]

    i_norm = jnp.sum(x * x, axis=1, keepdims=True)
    ones_row = jnp.ones((1, _EMB_DIM), dtype=jnp.float32)
    w_norm = jax.lax.dot_general(
        ones_row, w * w, (((1,), (1,)), ((), ())),
        preferred_element_type=jnp.float32)

    mm = jax.lax.dot_general(
        x, w, (((1,), (1,)), ((), ())),
        preferred_element_type=jnp.float32)
    dist = (i_norm + w_norm) - 2.0 * mm

    mn = jnp.min(dist, axis=1, keepdims=True)
    iota = jax.lax.broadcasted_iota(jnp.int32, (b, _NUM_EMB), 1)
    cand = jnp.where(dist == mn, iota, jnp.int32(_NUM_EMB))
    idx = jnp.min(cand, axis=1, keepdims=True)

    one_hot = (iota == idx).astype(jnp.float32)
    q = jax.lax.dot_general(
        one_hot, w, (((1,), (0,)), ((), ())),
        preferred_element_type=jnp.float32)

    out_ref[...] = x + (q - x)
    loss_ref[...] = jnp.sum((x - q) ** 2).reshape(1, 1, 1)


@functools.partial(jax.jit, static_argnames=())
def kernel(input, W):
    shape = input.shape
    flat = input.reshape(-1, shape[-1])
    n = flat.shape[0]
    grid = n // _BLOCK

    out, loss_parts = pl.pallas_call(
        _vq_block_kernel,
        grid=(grid,),
        in_specs=[
            pl.BlockSpec((_BLOCK, _EMB_DIM), lambda i: (i, 0)),
            pl.BlockSpec((_NUM_EMB, _EMB_DIM), lambda i: (0, 0)),
        ],
        out_specs=[
            pl.BlockSpec((_BLOCK, _EMB_DIM), lambda i: (i, 0)),
            pl.BlockSpec((1, 1, 1), lambda i: (i, 0, 0)),
        ],
        out_shape=[
            jax.ShapeDtypeStruct((n, _EMB_DIM), jnp.float32),
            jax.ShapeDtypeStruct((grid, 1, 1), jnp.float32),
        ],
    )(flat, W)

    e_latent_loss = jnp.sum(loss_parts) / jnp.float32(n * _EMB_DIM)
    loss = 0.25 * e_latent_loss
    return (out.reshape(shape), loss.reshape(()))
